# baseline (speedup 1.0000x reference)
import functools

import jax
import jax.numpy as jnp
from jax import lax
from jax.experimental import pallas as pl
from jax.experimental.pallas import tpu as pltpu
from jax.experimental.pallas import tpu_sc as plsc

B = 16384
D = 64
NC = 2
NS = 16
NW = NC * NS
BPW = B // NW
L = 16

_mesh = plsc.VectorSubcoreMesh(core_axis_name="c", subcore_axis_name="s")


def _neg_sqrt(x):
    i = plsc.bitcast(x, jnp.int32)
    i = jnp.int32(0x5F3759DF) - lax.shift_right_logical(i, 1)
    y = plsc.bitcast(i, jnp.float32)
    for _ in range(3):
        y = y * (1.5 - 0.5 * x * y * y)
    return jnp.where(x > 0.0, -x * y, 0.0)


@functools.partial(
    pl.kernel,
    mesh=_mesh,
    compiler_params=pltpu.CompilerParams(
        needs_layout_passes=False, use_tc_tiling_on_sc=False),
    out_type=jax.ShapeDtypeStruct((B,), jnp.float32),
    scratch_types=[
        pltpu.VMEM((BPW,), jnp.int32),
        pltpu.VMEM((BPW,), jnp.int32),
        pltpu.VMEM((BPW,), jnp.int32),
        pltpu.VMEM((BPW, D), jnp.float32),
        pltpu.VMEM((BPW, D), jnp.float32),
        pltpu.VMEM((BPW, D), jnp.float32),
        pltpu.VMEM((BPW,), jnp.float32),
        pltpu.SemaphoreType.DMA,
        pltpu.SemaphoreType.DMA,
        pltpu.SemaphoreType.DMA,
    ],
)
def _transe_sc(ent_hbm, rel_hbm, heads_hbm, rels_hbm, tails_hbm, out_hbm,
               hidx, ridx, tidx, hrow, rrow, trow, outv,
               sem_h, sem_r, sem_t):
    wid = lax.axis_index("s") * NC + lax.axis_index("c")
    base = wid * BPW

    pltpu.sync_copy(heads_hbm.at[pl.ds(base, BPW)], hidx)
    pltpu.sync_copy(rels_hbm.at[pl.ds(base, BPW)], ridx)
    pltpu.sync_copy(tails_hbm.at[pl.ds(base, BPW)], tidx)

    cp_h = pltpu.async_copy(ent_hbm.at[hidx], hrow, sem_h)
    cp_r = pltpu.async_copy(rel_hbm.at[ridx], rrow, sem_r)
    cp_t = pltpu.async_copy(ent_hbm.at[tidx], trow, sem_t)
    cp_h.wait()
    cp_r.wait()
    cp_t.wait()

    lanes = lax.iota(jnp.int32, L)

    def body(g, carry):
        packed = jnp.zeros((L,), jnp.float32)
        for j in range(L):
            item = g * L + j
            acc = jnp.zeros((L,), jnp.float32)
            for c in range(D // L):
                sl = pl.ds(c * L, L)
                dv = hrow[item, sl] + rrow[item, sl] - trow[item, sl]
                acc = acc + dv * dv
            packed = jnp.where(lanes == j, jnp.sum(acc), packed)
        outv[pl.ds(g * L, L)] = _neg_sqrt(packed)
        return carry

    lax.fori_loop(0, BPW // L, body, 0)
    pltpu.sync_copy(outv, out_hbm.at[pl.ds(base, BPW)])


def kernel(entity_emb, relation_emb, heads, relations, tails):
    return _transe_sc(entity_emb, relation_emb,
                      heads.astype(jnp.int32),
                      relations.astype(jnp.int32),
                      tails.astype(jnp.int32))


# speedup vs baseline: 1.6705x; 1.6705x over previous
import functools

import jax
import jax.numpy as jnp
from jax import lax
from jax.experimental import pallas as pl
from jax.experimental.pallas import tpu as pltpu
from jax.experimental.pallas import tpu_sc as plsc

B = 16384
D = 64
NC = 2
NS = 16
NW = NC * NS
BPW = B // NW
L = 16
CH = 128

_mesh = plsc.VectorSubcoreMesh(core_axis_name="c", subcore_axis_name="s")


def _neg_sqrt(x):
    i = plsc.bitcast(x, jnp.int32)
    i = jnp.int32(0x5F3759DF) - lax.shift_right_logical(i, 1)
    y = plsc.bitcast(i, jnp.float32)
    for _ in range(3):
        y = y * (1.5 - 0.5 * x * y * y)
    return jnp.where(x > 0.0, -x * y, 0.0)


@functools.partial(
    pl.kernel,
    mesh=_mesh,
    compiler_params=pltpu.CompilerParams(
        needs_layout_passes=False, use_tc_tiling_on_sc=True),
    out_type=jax.ShapeDtypeStruct((B,), jnp.float32),
    scratch_types=[
        pltpu.VMEM((BPW,), jnp.int32),
        pltpu.VMEM((BPW,), jnp.int32),
        pltpu.VMEM((BPW,), jnp.int32),
        pltpu.VMEM((CH, D), jnp.float32),
        pltpu.VMEM((CH, D), jnp.float32),
        pltpu.VMEM((CH, D), jnp.float32),
        pltpu.VMEM((BPW,), jnp.float32),
        pltpu.SemaphoreType.DMA,
        pltpu.SemaphoreType.DMA,
        pltpu.SemaphoreType.DMA,
    ],
)
def _transe_sc(ent_hbm, rel_hbm, heads_hbm, rels_hbm, tails_hbm, out_hbm,
               hidx, ridx, tidx, hrow, rrow, trow, outv,
               sem_h, sem_r, sem_t):
    wid = lax.axis_index("s") * NC + lax.axis_index("c")
    base = wid * BPW

    pltpu.sync_copy(heads_hbm.at[pl.ds(base, BPW)], hidx)
    pltpu.sync_copy(rels_hbm.at[pl.ds(base, BPW)], ridx)
    pltpu.sync_copy(tails_hbm.at[pl.ds(base, BPW)], tidx)

    lanes = lax.iota(jnp.int32, L)

    def chunk(ci, carry):
        def fetch(g, c2):
            hv = hidx[pl.ds(ci * CH + g * L, L)]
            rv = ridx[pl.ds(ci * CH + g * L, L)]
            tv = tidx[pl.ds(ci * CH + g * L, L)]
            for j in range(L):
                i = g * L + j
                pltpu.async_copy(ent_hbm.at[hv[j]], hrow.at[i], sem_h)
                pltpu.async_copy(rel_hbm.at[rv[j]], rrow.at[i], sem_r)
                pltpu.async_copy(ent_hbm.at[tv[j]], trow.at[i], sem_t)
            return c2

        lax.fori_loop(0, CH // L, fetch, 0)
        pltpu.make_async_copy(ent_hbm.at[pl.ds(0, CH)], hrow, sem_h).wait()
        pltpu.make_async_copy(rel_hbm.at[pl.ds(0, CH)], rrow, sem_r).wait()
        pltpu.make_async_copy(ent_hbm.at[pl.ds(0, CH)], trow, sem_t).wait()

        def body(g, c2):
            packed = jnp.zeros((L,), jnp.float32)
            for j in range(L):
                item = g * L + j
                acc = jnp.zeros((L,), jnp.float32)
                for c in range(D // L):
                    sl = pl.ds(c * L, L)
                    dv = hrow[item, sl] + rrow[item, sl] - trow[item, sl]
                    acc = acc + dv * dv
                packed = jnp.where(lanes == j, jnp.sum(acc), packed)
            outv[pl.ds(ci * CH + g * L, L)] = _neg_sqrt(packed)
            return c2

        lax.fori_loop(0, CH // L, body, 0)
        return carry

    lax.fori_loop(0, BPW // CH, chunk, 0)
    pltpu.sync_copy(outv, out_hbm.at[pl.ds(base, BPW)])


def kernel(entity_emb, relation_emb, heads, relations, tails):
    return _transe_sc(entity_emb, relation_emb,
                      heads.astype(jnp.int32),
                      relations.astype(jnp.int32),
                      tails.astype(jnp.int32))
